# baseline (speedup 1.0000x reference)
import functools

import jax
import jax.numpy as jnp
from jax import lax
from jax.experimental import pallas as pl
from jax.experimental.pallas import tpu as pltpu
from jax.experimental.pallas import tpu_sc as plsc

B, L = 4096, 200
N = B * L
DIMS = (32, 16, 16, 16, 16, 32)
TOTAL = 112
PROJ = 128

SLICE_SIZES = (245760, 245760, 245760, 81920)
NC, NS = 2, 16
NW = NC * NS
CHUNK = 128
KROW = CHUNK // 128
IDX_ROWS = N // 128
REP = 128

_PACK = ((0, 0, 0), (0, 16, 16), (1, 0, 32), (2, 0, 48), (3, 0, 64),
         (4, 0, 80), (5, 0, 96), (5, 16, 112))


def _sc_gather_concat(idxs, tabs, pos0, npos):
    per_w = npos // NW
    steps = per_w // CHUNK
    assert steps % 2 == 0 and steps >= 4
    mesh = plsc.VectorSubcoreMesh(core_axis_name="c", subcore_axis_name="s")
    out_type = jax.ShapeDtypeStruct((npos, PROJ), jnp.float32)
    scratch = []
    for _b in range(2):
        for _f in range(6):
            scratch.append(pltpu.VMEM((KROW, 128), jnp.int32))
        for d in DIMS:
            scratch.append(pltpu.VMEM((CHUNK, d), jnp.float32))
    scratch.append(pltpu.VMEM((CHUNK, PROJ), jnp.float32))
    for _s in range(5):
        scratch.append(pltpu.SemaphoreType.DMA)

    @functools.partial(pl.kernel, out_type=out_type, mesh=mesh,
                       scratch_types=scratch,
                       compiler_params=pltpu.CompilerParams(
                           use_tc_tiling_on_sc=False))
    def body(i0, i1, i2, i3, i4, i5, t0, t1, t2, t3, t4, t5, out,
             a0, a1, a2, a3, a4, a5, ra0, ra1, ra2, ra3, ra4, ra5,
             b0, b1, b2, b3, b4, b5, rb0, rb1, rb2, rb3, rb4, rb5,
             cat, semi0, semi1, semg0, semg1, semw):
        idx_refs = (i0, i1, i2, i3, i4, i5)
        tab_refs = (t0, t1, t2, t3, t4, t5)
        ibufs = ((a0, a1, a2, a3, a4, a5), (b0, b1, b2, b3, b4, b5))
        rbufs = ((ra0, ra1, ra2, ra3, ra4, ra5),
                 (rb0, rb1, rb2, rb3, rb4, rb5))
        semi = (semi0, semi1)
        semg = (semg0, semg1)
        wid = lax.axis_index("s") * NC + lax.axis_index("c")
        row_base = pos0 // 128 + wid * (per_w // 128)
        pos_base = wid * per_w

        def fire_idx(g, sl):
            g = jnp.minimum(g, steps - 1)
            for f in range(6):
                pltpu.async_copy(
                    idx_refs[f].at[pl.ds(row_base + g * KROW, KROW)],
                    ibufs[sl][f], semi[sl])

        def drain_idx(sl):
            for f in range(6):
                pltpu.make_async_copy(
                    idx_refs[f].at[pl.ds(row_base, KROW)],
                    ibufs[sl][f], semi[sl]).wait()

        def fire_gathers(sl):
            for f in range(6):
                for j in range(KROW):
                    pltpu.async_copy(
                        tab_refs[f].at[ibufs[sl][f].at[j]],
                        rbufs[sl][f].at[pl.ds(j * 128, 128)], semg[sl])

        def wait_gathers(sl):
            for f in range(6):
                for j in range(KROW):
                    pltpu.make_async_copy(
                        tab_refs[f].at[ibufs[sl][f].at[j]],
                        rbufs[sl][f].at[pl.ds(j * 128, 128)],
                        semg[sl]).wait()

        def repack(sl):
            @plsc.parallel_loop(0, CHUNK, unroll=4)
            def row(r):
                for f, soff, doff in _PACK:
                    cat[r, pl.ds(doff, 16)] = rbufs[sl][f][r, pl.ds(soff, 16)]

        def fire_write(g):
            pltpu.async_copy(
                cat, out.at[pl.ds(pos_base + g * CHUNK, CHUNK)], semw)

        def drain_write():
            pltpu.make_async_copy(
                cat, out.at[pl.ds(pos_base, CHUNK)], semw).wait()

        def half(c, sl, fire_next, first_w):
            other = 1 - sl
            if fire_next:
                drain_idx(other)
                fire_gathers(other)
            wait_gathers(sl)
            fire_idx(c + 2, sl)
            if not first_w:
                drain_write()
            repack(sl)
            fire_write(c)

        fire_idx(jnp.int32(0), 0)
        fire_idx(jnp.int32(1), 1)
        drain_idx(0)
        fire_gathers(0)
        half(jnp.int32(0), 0, True, True)
        half(jnp.int32(1), 1, True, False)

        def step(k, carry):
            half(2 * k, 0, True, False)
            half(2 * k + 1, 1, True, False)
            return carry

        lax.fori_loop(1, steps // 2 - 1, step, 0)

        half(jnp.int32(steps - 2), 0, True, False)
        half(jnp.int32(steps - 1), 1, False, False)

        drain_write()
        drain_idx(0)
        drain_idx(1)

    return body(*idxs, *tabs)


def _tc_project(cat_s, w_t_pad, b2, pos0, npos, prev):
    BLK = 4096
    grid = (npos // BLK,)
    off = pos0 // BLK

    def body(*refs):
        x, wt, bb, out = refs[0], refs[1], refs[2], refs[-1]
        out[...] = jnp.dot(x[...], wt[...],
                           preferred_element_type=jnp.float32) + bb[...]

    in_specs = [pl.BlockSpec((BLK, PROJ), lambda i: (i, 0)),
                pl.BlockSpec((PROJ, PROJ), lambda i: (0, 0)),
                pl.BlockSpec((1, PROJ), lambda i: (0, 0))]
    args = [cat_s, w_t_pad, b2]
    aliases = {}
    if prev is not None:
        in_specs.append(pl.BlockSpec(memory_space=pl.ANY))
        args.append(prev)
        aliases = {3: 0}
    return pl.pallas_call(
        body, grid=grid, in_specs=in_specs,
        out_specs=pl.BlockSpec((BLK, PROJ), lambda i: (i + off, 0)),
        out_shape=jax.ShapeDtypeStruct((N, PROJ), jnp.float32),
        input_output_aliases=aliases,
    )(*args)


def kernel(merchant_id, category, mcc, country, currency, hour_of_day,
           emb_merchant_id, emb_category, emb_mcc, emb_country, emb_currency,
           emb_hour_of_day, W, b):
    rep_off = jnp.arange(N, dtype=jnp.int32) // (N // REP)
    hour_pad = jnp.pad(emb_hour_of_day, ((0, 0), (0, 16)))
    idx_flat = (merchant_id.reshape(N), category.reshape(N), mcc.reshape(N),
                country.reshape(N) + rep_off * 200,
                currency.reshape(N) + rep_off * 50,
                hour_of_day.reshape(N) + rep_off * 24)
    idxs = [a.reshape(IDX_ROWS, 128) for a in idx_flat]
    tabs = (emb_merchant_id, emb_category, emb_mcc,
            jnp.tile(emb_country, (REP, 1)), jnp.tile(emb_currency, (REP, 1)),
            jnp.tile(hour_pad, (REP, 1)))
    w_t_pad = jnp.zeros((PROJ, PROJ), jnp.float32).at[:TOTAL].set(W.T)
    b2 = b.reshape(1, PROJ)

    offs = [sum(SLICE_SIZES[:i]) for i in range(len(SLICE_SIZES))]
    cats = [_sc_gather_concat(idxs, tabs, p0, np_)
            for p0, np_ in zip(offs, SLICE_SIZES)]
    out = None
    for cat_s, p0, np_ in zip(cats, offs, SLICE_SIZES):
        out = _tc_project(cat_s, w_t_pad, b2, p0, np_, out)
    return out.reshape(B, L, PROJ)


# speedup vs baseline: 1.0093x; 1.0093x over previous
import functools

import jax
import jax.numpy as jnp
from jax import lax
from jax.experimental import pallas as pl
from jax.experimental.pallas import tpu as pltpu
from jax.experimental.pallas import tpu_sc as plsc

B, L = 4096, 200
N = B * L
DIMS = (32, 16, 16, 16, 16, 32)
TOTAL = 112
PROJ = 128

SLICE_SIZES = (81920, 245760, 245760, 163840, 81920)
NC, NS = 2, 16
NW = NC * NS
CHUNK = 128
KROW = CHUNK // 128
IDX_ROWS = N // 128
REP = 128

_PACK = ((0, 0, 0), (0, 16, 16), (1, 0, 32), (2, 0, 48), (3, 0, 64),
         (4, 0, 80), (5, 0, 96), (5, 16, 112))


def _sc_gather_concat(idxs, tabs, pos0, npos):
    per_w = npos // NW
    steps = per_w // CHUNK
    assert steps % 2 == 0 and steps >= 4
    mesh = plsc.VectorSubcoreMesh(core_axis_name="c", subcore_axis_name="s")
    out_type = jax.ShapeDtypeStruct((npos, PROJ), jnp.float32)
    scratch = []
    for _b in range(2):
        for _f in range(6):
            scratch.append(pltpu.VMEM((KROW, 128), jnp.int32))
        for d in DIMS:
            scratch.append(pltpu.VMEM((CHUNK, d), jnp.float32))
    scratch.append(pltpu.VMEM((CHUNK, PROJ), jnp.float32))
    for _s in range(5):
        scratch.append(pltpu.SemaphoreType.DMA)

    @functools.partial(pl.kernel, out_type=out_type, mesh=mesh,
                       scratch_types=scratch,
                       compiler_params=pltpu.CompilerParams(
                           use_tc_tiling_on_sc=False))
    def body(i0, i1, i2, i3, i4, i5, t0, t1, t2, t3, t4, t5, out,
             a0, a1, a2, a3, a4, a5, ra0, ra1, ra2, ra3, ra4, ra5,
             b0, b1, b2, b3, b4, b5, rb0, rb1, rb2, rb3, rb4, rb5,
             cat, semi0, semi1, semg0, semg1, semw):
        idx_refs = (i0, i1, i2, i3, i4, i5)
        tab_refs = (t0, t1, t2, t3, t4, t5)
        ibufs = ((a0, a1, a2, a3, a4, a5), (b0, b1, b2, b3, b4, b5))
        rbufs = ((ra0, ra1, ra2, ra3, ra4, ra5),
                 (rb0, rb1, rb2, rb3, rb4, rb5))
        semi = (semi0, semi1)
        semg = (semg0, semg1)
        wid = lax.axis_index("s") * NC + lax.axis_index("c")
        row_base = pos0 // 128 + wid * (per_w // 128)
        pos_base = wid * per_w

        def fire_idx(g, sl):
            g = jnp.minimum(g, steps - 1)
            for f in range(6):
                pltpu.async_copy(
                    idx_refs[f].at[pl.ds(row_base + g * KROW, KROW)],
                    ibufs[sl][f], semi[sl])

        def drain_idx(sl):
            for f in range(6):
                pltpu.make_async_copy(
                    idx_refs[f].at[pl.ds(row_base, KROW)],
                    ibufs[sl][f], semi[sl]).wait()

        def fire_gathers(sl):
            for f in range(6):
                for j in range(KROW):
                    pltpu.async_copy(
                        tab_refs[f].at[ibufs[sl][f].at[j]],
                        rbufs[sl][f].at[pl.ds(j * 128, 128)], semg[sl])

        def wait_gathers(sl):
            for f in range(6):
                for j in range(KROW):
                    pltpu.make_async_copy(
                        tab_refs[f].at[ibufs[sl][f].at[j]],
                        rbufs[sl][f].at[pl.ds(j * 128, 128)],
                        semg[sl]).wait()

        def repack(sl):
            @plsc.parallel_loop(0, CHUNK, unroll=4)
            def row(r):
                for f, soff, doff in _PACK:
                    cat[r, pl.ds(doff, 16)] = rbufs[sl][f][r, pl.ds(soff, 16)]

        def fire_write(g):
            pltpu.async_copy(
                cat, out.at[pl.ds(pos_base + g * CHUNK, CHUNK)], semw)

        def drain_write():
            pltpu.make_async_copy(
                cat, out.at[pl.ds(pos_base, CHUNK)], semw).wait()

        def half(c, sl, fire_next, first_w):
            other = 1 - sl
            if fire_next:
                drain_idx(other)
                fire_gathers(other)
            wait_gathers(sl)
            fire_idx(c + 2, sl)
            if not first_w:
                drain_write()
            repack(sl)
            fire_write(c)

        fire_idx(jnp.int32(0), 0)
        fire_idx(jnp.int32(1), 1)
        drain_idx(0)
        fire_gathers(0)
        half(jnp.int32(0), 0, True, True)
        half(jnp.int32(1), 1, True, False)

        def step(k, carry):
            half(2 * k, 0, True, False)
            half(2 * k + 1, 1, True, False)
            return carry

        lax.fori_loop(1, steps // 2 - 1, step, 0)

        half(jnp.int32(steps - 2), 0, True, False)
        half(jnp.int32(steps - 1), 1, False, False)

        drain_write()
        drain_idx(0)
        drain_idx(1)

    return body(*idxs, *tabs)


def _tc_project(cat_s, w_t_pad, b2, pos0, npos, prev):
    BLK = 4096
    grid = (npos // BLK,)
    off = pos0 // BLK

    def body(*refs):
        x, wt, bb, out = refs[0], refs[1], refs[2], refs[-1]
        out[...] = jnp.dot(x[...], wt[...],
                           preferred_element_type=jnp.float32) + bb[...]

    in_specs = [pl.BlockSpec((BLK, PROJ), lambda i: (i, 0)),
                pl.BlockSpec((PROJ, PROJ), lambda i: (0, 0)),
                pl.BlockSpec((1, PROJ), lambda i: (0, 0))]
    args = [cat_s, w_t_pad, b2]
    aliases = {}
    if prev is not None:
        in_specs.append(pl.BlockSpec(memory_space=pl.ANY))
        args.append(prev)
        aliases = {3: 0}
    return pl.pallas_call(
        body, grid=grid, in_specs=in_specs,
        out_specs=pl.BlockSpec((BLK, PROJ), lambda i: (i + off, 0)),
        out_shape=jax.ShapeDtypeStruct((N, PROJ), jnp.float32),
        input_output_aliases=aliases,
    )(*args)


def kernel(merchant_id, category, mcc, country, currency, hour_of_day,
           emb_merchant_id, emb_category, emb_mcc, emb_country, emb_currency,
           emb_hour_of_day, W, b):
    rep_off = jnp.arange(N, dtype=jnp.int32) // (N // REP)
    hour_pad = jnp.pad(emb_hour_of_day, ((0, 0), (0, 16)))
    idx_flat = (merchant_id.reshape(N), category.reshape(N), mcc.reshape(N),
                country.reshape(N) + rep_off * 200,
                currency.reshape(N) + rep_off * 50,
                hour_of_day.reshape(N) + rep_off * 24)
    idxs = [a.reshape(IDX_ROWS, 128) for a in idx_flat]
    tabs = (emb_merchant_id, emb_category, emb_mcc,
            jnp.tile(emb_country, (REP, 1)), jnp.tile(emb_currency, (REP, 1)),
            jnp.tile(hour_pad, (REP, 1)))
    w_t_pad = jnp.zeros((PROJ, PROJ), jnp.float32).at[:TOTAL].set(W.T)
    b2 = b.reshape(1, PROJ)

    offs = [sum(SLICE_SIZES[:i]) for i in range(len(SLICE_SIZES))]
    cats = [_sc_gather_concat(idxs, tabs, p0, np_)
            for p0, np_ in zip(offs, SLICE_SIZES)]
    out = None
    for cat_s, p0, np_ in zip(cats, offs, SLICE_SIZES):
        out = _tc_project(cat_s, w_t_pad, b2, p0, np_, out)
    return out.reshape(B, L, PROJ)
